# baseline (speedup 1.0000x reference)
import functools

import jax
import jax.numpy as jnp
from jax.experimental import pallas as pl
from jax.experimental.pallas import tpu as pltpu

EPS = 1e-08


def _body(attn_ref, feat_ref, hc_ref, wn_ref, bn_ref, wnb_ref, bnb_ref,
          alpha_ref, g_ref, b_ref, out_ref, pre_ref, acc_ref,
          *, bm, bm2, nsteps, n):
    s = pl.program_id(0)

    @pl.when(s < nsteps)
    def _():
        gate = jax.nn.sigmoid(alpha_ref[0, 0])
        nbf = jnp.dot(attn_ref[...], feat_ref[...],
                      preferred_element_type=jnp.float32)
        nb = jnp.dot(nbf, wnb_ref[...],
                     preferred_element_type=jnp.float32) + bnb_ref[...]
        fi = feat_ref[pl.ds(s * bm, bm), :]
        node = jnp.dot(fi, wn_ref[...],
                       preferred_element_type=jnp.float32) + bn_ref[...]
        pre = node + (gate * hc_ref[...]) * nb
        pre_ref[pl.ds(s * bm, bm), :] = pre
        blk = jnp.concatenate(
            [jnp.sum(pre, axis=0, keepdims=True),
             jnp.sum(pre * pre, axis=0, keepdims=True)], axis=0)

        @pl.when(s == 0)
        def _():
            acc_ref[...] = blk

        @pl.when(s > 0)
        def _():
            acc_ref[...] += blk

    @pl.when(s >= nsteps)
    def _():
        j = s - nsteps
        x = pre_ref[pl.ds(j * bm2, bm2), :]
        mean = acc_ref[0:1, :] / n
        var = (acc_ref[1:2, :] - n * mean * mean) / (n - 1)
        std = jnp.sqrt(jnp.maximum(var, 0.0))
        y = g_ref[...] * ((x - mean) / (std + EPS)) + b_ref[...]
        out_ref[...] = jnp.maximum(y, 0.0)


def kernel(feat, edge_index, attn_matrix, has_cn, W_node, b_node, W_nb, b_nb,
           alpha, gamma, beta):
    n, d_in = feat.shape
    d_out = W_node.shape[1]
    bm = 400
    nsteps = n // bm
    bm2 = 1000 if n % 1000 == 0 else n
    nphase2 = n // bm2
    assert n % bm == 0
    last = nsteps - 1
    const = lambda s: (0, 0)
    irow = lambda s: (jnp.minimum(s, last), 0)
    out = pl.pallas_call(
        functools.partial(_body, bm=bm, bm2=bm2, nsteps=nsteps, n=n),
        grid=(nsteps + nphase2,),
        in_specs=[
            pl.BlockSpec((bm, n), irow),
            pl.BlockSpec((n, d_in), const),
            pl.BlockSpec((bm, 1), irow),
            pl.BlockSpec((d_in, d_out), const),
            pl.BlockSpec((1, d_out), const),
            pl.BlockSpec((d_in, d_out), const),
            pl.BlockSpec((1, d_out), const),
            pl.BlockSpec((1, 1), const),
            pl.BlockSpec((1, d_out), const),
            pl.BlockSpec((1, d_out), const),
        ],
        out_specs=pl.BlockSpec(
            (bm2, d_out), lambda s: (jnp.maximum(s - nsteps, 0), 0)),
        out_shape=jax.ShapeDtypeStruct((n, d_out), jnp.float32),
        scratch_shapes=[
            pltpu.VMEM((n, d_out), jnp.float32),
            pltpu.VMEM((2, d_out), jnp.float32),
        ],
        compiler_params=pltpu.CompilerParams(
            dimension_semantics=("arbitrary",)),
    )(attn_matrix, feat, has_cn, W_node,
      b_node.reshape(1, d_out), W_nb, b_nb.reshape(1, d_out),
      alpha.reshape(1, 1), gamma.reshape(1, d_out), beta.reshape(1, d_out))
    return out


# speedup vs baseline: 1.0124x; 1.0124x over previous
import functools

import jax
import jax.numpy as jnp
from jax.experimental import pallas as pl
from jax.experimental.pallas import tpu as pltpu

EPS = 1e-08


def _body(attn_ref, feat_ref, hc_ref, wn_ref, bn_ref, wnb_ref, bnb_ref,
          alpha_ref, g_ref, b_ref, out_ref, pre_ref, acc_ref,
          *, bm, bm2, nsteps, n):
    s = pl.program_id(0)

    @pl.when(s < nsteps)
    def _():
        gate = jax.nn.sigmoid(alpha_ref[0, 0])
        nbf = jnp.dot(attn_ref[...], feat_ref[...],
                      preferred_element_type=jnp.float32)
        nb = jnp.dot(nbf, wnb_ref[...],
                     preferred_element_type=jnp.float32) + bnb_ref[...]
        fi = feat_ref[pl.ds(s * bm, bm), :]
        node = jnp.dot(fi, wn_ref[...],
                       preferred_element_type=jnp.float32) + bn_ref[...]
        pre = node + (gate * hc_ref[...]) * nb
        pre_ref[pl.ds(s * bm, bm), :] = pre
        blk = jnp.concatenate(
            [jnp.sum(pre, axis=0, keepdims=True),
             jnp.sum(pre * pre, axis=0, keepdims=True)], axis=0)

        @pl.when(s == 0)
        def _():
            acc_ref[...] = blk

        @pl.when(s > 0)
        def _():
            acc_ref[...] += blk

    @pl.when(s >= nsteps)
    def _():
        j = s - nsteps
        x = pre_ref[pl.ds(j * bm2, bm2), :]
        mean = acc_ref[0:1, :] / n
        var = (acc_ref[1:2, :] - n * mean * mean) / (n - 1)
        std = jnp.sqrt(jnp.maximum(var, 0.0))
        y = g_ref[...] * ((x - mean) / (std + EPS)) + b_ref[...]
        out_ref[...] = jnp.maximum(y, 0.0)


def kernel(feat, edge_index, attn_matrix, has_cn, W_node, b_node, W_nb, b_nb,
           alpha, gamma, beta):
    n, d_in = feat.shape
    d_out = W_node.shape[1]
    bm = 400
    nsteps = n // bm
    bm2 = 2000 if n % 2000 == 0 else n
    nphase2 = n // bm2
    assert n % bm == 0
    last = nsteps - 1
    const = lambda s: (0, 0)
    irow = lambda s: (jnp.minimum(s, last), 0)
    out = pl.pallas_call(
        functools.partial(_body, bm=bm, bm2=bm2, nsteps=nsteps, n=n),
        grid=(nsteps + nphase2,),
        in_specs=[
            pl.BlockSpec((bm, n), irow),
            pl.BlockSpec((n, d_in), const),
            pl.BlockSpec((bm, 1), irow),
            pl.BlockSpec((d_in, d_out), const),
            pl.BlockSpec((1, d_out), const),
            pl.BlockSpec((d_in, d_out), const),
            pl.BlockSpec((1, d_out), const),
            pl.BlockSpec((1, 1), const),
            pl.BlockSpec((1, d_out), const),
            pl.BlockSpec((1, d_out), const),
        ],
        out_specs=pl.BlockSpec(
            (bm2, d_out), lambda s: (jnp.maximum(s - nsteps, 0), 0)),
        out_shape=jax.ShapeDtypeStruct((n, d_out), jnp.float32),
        scratch_shapes=[
            pltpu.VMEM((n, d_out), jnp.float32),
            pltpu.VMEM((2, d_out), jnp.float32),
        ],
        compiler_params=pltpu.CompilerParams(
            dimension_semantics=("arbitrary",)),
    )(attn_matrix, feat, has_cn, W_node,
      b_node.reshape(1, d_out), W_nb, b_nb.reshape(1, d_out),
      alpha.reshape(1, 1), gamma.reshape(1, d_out), beta.reshape(1, d_out))
    return out
